# Initial kernel scaffold; baseline (speedup 1.0000x reference)
import jax
import jax.numpy as jnp
from jax.experimental import pallas as pl


def kernel(x, W1, b1, a_src, a_dst, W_edge, b_edge, edge_index):
    raise NotImplementedError("write your pallas kernel here")


# speedup vs baseline: 3.7930x; 3.7930x over previous
import functools

import jax
import jax.numpy as jnp
from jax import lax
from jax.experimental import pallas as pl
from jax.experimental.pallas import tpu as pltpu
from jax.experimental.pallas import tpu_sc as plsc

N = 10000
E = 160000
DIN = 128
H = 768
NR = 47
NRP = 128

NC, NS = 2, 16
NW = NC * NS
HC = 128
NCH = H // HC
EPW = E // NW
EPT = E // NS
CB = 80
NSH = 10240
SRT = NSH // NS
ZR = 128
EB = 40

_mesh = plsc.VectorSubcoreMesh(core_axis_name="c", subcore_axis_name="s")


def _stage_a_body(x_ref, w1_ref, b1_ref, aa_ref, h4_ref, esd_ref):
    c = pl.program_id(1)
    hb = jax.nn.gelu(
        jnp.dot(x_ref[...], w1_ref[0], preferred_element_type=jnp.float32)
        + b1_ref[0])
    h4_ref[...] = hb
    pe = jnp.dot(hb, aa_ref[...], preferred_element_type=jnp.float32)

    @pl.when(c == 0)
    def _():
        esd_ref[...] = pe

    @pl.when(c != 0)
    def _():
        esd_ref[...] += pe


def _stage_a(x, W1, b1, aa):
    rb = 1000
    return pl.pallas_call(
        _stage_a_body,
        grid=(N // rb, NCH),
        in_specs=[
            pl.BlockSpec((rb, DIN), lambda i, c: (i, 0)),
            pl.BlockSpec((1, DIN, HC), lambda i, c: (c, 0, 0)),
            pl.BlockSpec((1, 1, HC), lambda i, c: (c, 0, 0)),
            pl.BlockSpec((HC, 128), lambda i, c: (c, 0)),
        ],
        out_specs=[
            pl.BlockSpec((rb, HC), lambda i, c: (c * (N // rb) + i, 0)),
            pl.BlockSpec((rb, 128), lambda i, c: (i, 0)),
        ],
        out_shape=[
            jax.ShapeDtypeStruct((NCH * N, HC), jnp.float32),
            jax.ShapeDtypeStruct((N, 128), jnp.float32),
        ],
    )(x, W1.reshape(DIN, NCH, HC).transpose(1, 0, 2),
      b1.reshape(NCH, 1, HC), aa)


EPW16 = EPW + 16 - EPW % 16 if EPW % 16 else EPW


@functools.partial(
    pl.kernel,
    out_type=(jax.ShapeDtypeStruct((E,), jnp.float32),
              jax.ShapeDtypeStruct((NW * N,), jnp.float32)),
    mesh=_mesh,
    compiler_params=pltpu.CompilerParams(needs_layout_passes=False),
    scratch_types=[
        pltpu.VMEM((N,), jnp.float32),
        pltpu.VMEM((N,), jnp.float32),
        pltpu.VMEM((N,), jnp.float32),
        pltpu.VMEM((EPW16,), jnp.int32),
        pltpu.VMEM((EPW16,), jnp.int32),
        pltpu.VMEM((EPW16,), jnp.float32),
    ],
)
def _edge_u(es_hbm, ed_hbm, src_hbm, dst_hbm, u_hbm, den_hbm,
            es_v, ed_v, den_v, src_v, dst_v, u_v):
    cid = lax.axis_index("c")
    sid = lax.axis_index("s")
    wid = sid * NC + cid
    base = pl.multiple_of(wid * EPW, 8)
    pltpu.sync_copy(es_hbm, es_v)
    pltpu.sync_copy(ed_hbm, ed_v)
    pltpu.sync_copy(src_hbm.at[pl.ds(base, EPW)], src_v.at[pl.ds(0, EPW)])
    pltpu.sync_copy(dst_hbm.at[pl.ds(base, EPW)], dst_v.at[pl.ds(0, EPW)])

    def zero_body(i, _):
        den_v[pl.ds(i * 16, 16)] = jnp.zeros((16,), jnp.float32)
        return 0

    lax.fori_loop(0, N // 16, zero_body, 0)

    lane = lax.iota(jnp.int32, 16)

    def edge_body(t, _):
        off = t * 16
        valid = (off + lane) < EPW
        s16 = jnp.where(valid, src_v[pl.ds(off, 16)], 0)
        d16 = jnp.where(valid, dst_v[pl.ds(off, 16)], 0)
        a = plsc.load_gather(es_v, [s16])
        b = plsc.load_gather(ed_v, [d16])
        e16 = a + b
        e16 = jnp.where(e16 >= 0, e16, 0.2 * e16)
        u16 = jnp.where(valid, jnp.exp(e16), 0.0)
        u_v[pl.ds(off, 16)] = u16
        plsc.addupdate_scatter(den_v, [d16], u16)
        return 0

    lax.fori_loop(0, EPW16 // 16, edge_body, 0)

    pltpu.sync_copy(u_v.at[pl.ds(0, EPW)], u_hbm.at[pl.ds(base, EPW)])
    dbase = pl.multiple_of(wid * N, 8)
    pltpu.sync_copy(den_v, den_hbm.at[pl.ds(dbase, N)])


@functools.partial(
    pl.kernel,
    out_type=jax.ShapeDtypeStruct((NCH * N, HC), jnp.float32),
    mesh=_mesh,
    compiler_params=pltpu.CompilerParams(needs_layout_passes=False),
    scratch_types=[
        pltpu.VMEM((CB,), jnp.int32),
        pltpu.VMEM((CB,), jnp.int32),
        pltpu.VMEM((CB,), jnp.float32),
        pltpu.VMEM((CB, HC), jnp.float32),
        pltpu.VMEM((ZR, HC), jnp.float32),
        pltpu.VMEM_SHARED((NSH, HC), jnp.float32),
        pltpu.SemaphoreType.DMA,
    ],
)
def _aggregate(h4_hbm, u_hbm, src_hbm, dst_hbm, agg_hbm,
               sidx_v, didx_v, u_v, rows_v, zb_v, acc_sh, sem):
    cid = lax.axis_index("c")
    sid = lax.axis_index("s")

    def zfill(r, _):
        for j in range(HC // 16):
            zb_v[r, pl.ds(j * 16, 16)] = jnp.zeros((16,), jnp.float32)
        return 0

    lax.fori_loop(0, ZR, zfill, 0)

    for hc in range(NCH // NC):
        chunk = cid * (NCH // NC) + hc
        for k in range(SRT // ZR):
            pltpu.sync_copy(
                zb_v, acc_sh.at[pl.ds(sid * SRT + k * ZR, ZR)])
        plsc.subcore_barrier()

        def batch_body(t, _):
            eb = pl.multiple_of(sid * EPT + t * CB, 8)
            pltpu.sync_copy(src_hbm.at[pl.ds(eb, CB)], sidx_v)
            pltpu.sync_copy(dst_hbm.at[pl.ds(eb, CB)], didx_v)
            pltpu.sync_copy(u_hbm.at[pl.ds(eb, CB)], u_v)
            for j in range(CB // 16):
                sl = pl.ds(j * 16, 16)
                sidx_v[sl] = sidx_v[sl] + chunk * N
            pltpu.async_copy(h4_hbm.at[sidx_v], rows_v, sem).wait()

            def scale_body(i, _2):
                us = plsc.load_gather(u_v, [jnp.full((16,), i, jnp.int32)])
                for j in range(HC // 16):
                    sl = pl.ds(j * 16, 16)
                    rows_v[i, sl] = rows_v[i, sl] * us
                return 0

            lax.fori_loop(0, CB, scale_body, 0)
            pltpu.sync_copy(rows_v, acc_sh.at[didx_v], add=True)
            return 0

        lax.fori_loop(0, EPT // CB, batch_body, 0)
        plsc.subcore_barrier()
        last_rows = N - (NS - 1) * SRT

        @pl.when(sid < NS - 1)
        def _():
            pltpu.sync_copy(
                acc_sh.at[pl.ds(sid * SRT, SRT)],
                agg_hbm.at[pl.ds(chunk * N + sid * SRT, SRT)])

        @pl.when(sid == NS - 1)
        def _():
            pltpu.sync_copy(
                acc_sh.at[pl.ds((NS - 1) * SRT, last_rows)],
                agg_hbm.at[pl.ds(chunk * N + (NS - 1) * SRT, last_rows)])
        plsc.subcore_barrier()


def _proj_body(ag_ref, denp_ref, w1_ref, w2_ref, bp_ref, p1_ref, p2_ref):
    c = pl.program_id(1)
    den = jnp.sum(denp_ref[...], axis=1)
    inv = 1.0 / (den + 1e-9)
    sc = ag_ref[...] * inv[:, None]
    q1 = jnp.dot(sc, w1_ref[...], preferred_element_type=jnp.float32)
    q2 = jnp.dot(sc, w2_ref[...], preferred_element_type=jnp.float32)

    @pl.when(c == 0)
    def _():
        p1_ref[...] = q1 + bp_ref[...]
        p2_ref[...] = q2

    @pl.when(c != 0)
    def _():
        p1_ref[...] += q1
        p2_ref[...] += q2


def _proj(aggu, denp, Wp, bp):
    rb = 1000
    return pl.pallas_call(
        _proj_body,
        grid=(N // rb, NCH),
        in_specs=[
            pl.BlockSpec((rb, HC), lambda i, c: (c * (N // rb) + i, 0)),
            pl.BlockSpec((rb, NW), lambda i, c: (i, 0)),
            pl.BlockSpec((HC, NRP), lambda i, c: (c, 0)),
            pl.BlockSpec((HC, NRP), lambda i, c: (NCH + c, 0)),
            pl.BlockSpec((NRP,), lambda i, c: (0,)),
        ],
        out_specs=[
            pl.BlockSpec((rb, NRP), lambda i, c: (i, 0)),
            pl.BlockSpec((rb, NRP), lambda i, c: (i, 0)),
        ],
        out_shape=[
            jax.ShapeDtypeStruct((N, NRP), jnp.float32),
            jax.ShapeDtypeStruct((N, NRP), jnp.float32),
        ],
    )(aggu, denp, Wp, Wp, bp)


@functools.partial(
    pl.kernel,
    out_type=jax.ShapeDtypeStruct((E, NRP), jnp.float32),
    mesh=_mesh,
    compiler_params=pltpu.CompilerParams(needs_layout_passes=False),
    scratch_types=[
        pltpu.VMEM((EB,), jnp.int32),
        pltpu.VMEM((EB,), jnp.int32),
        pltpu.VMEM((EB, NRP), jnp.float32),
        pltpu.VMEM((EB, NRP), jnp.float32),
        pltpu.SemaphoreType.DMA,
    ],
)
def _edge_out(p1_hbm, p2_hbm, src_hbm, dst_hbm, lp_hbm,
              i1_v, i2_v, a_v, b_v, sem):
    cid = lax.axis_index("c")
    sid = lax.axis_index("s")
    wid = sid * NC + cid
    base = wid * EPW

    def batch_body(t, _):
        eb = pl.multiple_of(base + t * EB, 8)
        pltpu.sync_copy(src_hbm.at[pl.ds(eb, EB)], i1_v)
        pltpu.sync_copy(dst_hbm.at[pl.ds(eb, EB)], i2_v)
        pltpu.async_copy(p1_hbm.at[i1_v], a_v, sem).wait()
        pltpu.async_copy(p2_hbm.at[i2_v], b_v, sem).wait()

        def add_body(i, _2):
            for j in range(NRP // 16):
                sl = pl.ds(j * 16, 16)
                a_v[i, sl] = a_v[i, sl] + b_v[i, sl]
            return 0

        lax.fori_loop(0, EB, add_body, 0)
        pltpu.sync_copy(a_v, lp_hbm.at[pl.ds(eb, EB)])
        return 0

    lax.fori_loop(0, EPW // EB, batch_body, 0)


def kernel(x, W1, b1, a_src, a_dst, W_edge, b_edge, edge_index):
    src = edge_index[0]
    dst = edge_index[1]
    aa = jnp.concatenate(
        [a_src[:, None], a_dst[:, None], jnp.zeros((H, 126), jnp.float32)],
        axis=1)
    h4, esd = _stage_a(x, W1, b1, aa)
    es = esd[:, 0]
    ed = esd[:, 1]
    u, denp = _edge_u(es, ed, src, dst)
    aggu = _aggregate(h4, u, src, dst)
    Wp = jnp.pad(W_edge, ((0, 0), (0, NRP - NR)))
    bp = jnp.pad(b_edge, (0, NRP - NR))
    P1, P2 = _proj(aggu, denp.reshape(NW, N).T, Wp, bp)
    logits_p = _edge_out(P1, P2, src, dst)
    return logits_p[:, :NR]


# speedup vs baseline: 4.7189x; 1.2441x over previous
import functools

import jax
import jax.numpy as jnp
from jax import lax
from jax.experimental import pallas as pl
from jax.experimental.pallas import tpu as pltpu
from jax.experimental.pallas import tpu_sc as plsc

N = 10000
E = 160000
DIN = 128
H = 768
NR = 47
NRP = 128

NC, NS = 2, 16
NW = NC * NS
HC = 128
NCH = H // HC
EPW = E // NW
EP = 163840
EPT = EP // NS
CB = 128
NBUF = 2
NB = EPT // CB
NG = NB // NBUF
NSH = 10240
SRT = NSH // NS
ZR = 128
EB = 40

_mesh = plsc.VectorSubcoreMesh(core_axis_name="c", subcore_axis_name="s")


def _stage_a_body(x_ref, w1_ref, b1_ref, aa_ref, h4_ref, esd_ref):
    c = pl.program_id(1)
    hb = jax.nn.gelu(
        jnp.dot(x_ref[...], w1_ref[0], preferred_element_type=jnp.float32)
        + b1_ref[0])
    h4_ref[...] = hb
    pe = jnp.dot(hb, aa_ref[...], preferred_element_type=jnp.float32)

    @pl.when(c == 0)
    def _():
        esd_ref[...] = pe

    @pl.when(c != 0)
    def _():
        esd_ref[...] += pe


def _stage_a(x, W1, b1, aa):
    rb = 1000
    return pl.pallas_call(
        _stage_a_body,
        grid=(N // rb, NCH),
        in_specs=[
            pl.BlockSpec((rb, DIN), lambda i, c: (i, 0)),
            pl.BlockSpec((1, DIN, HC), lambda i, c: (c, 0, 0)),
            pl.BlockSpec((1, 1, HC), lambda i, c: (c, 0, 0)),
            pl.BlockSpec((HC, 128), lambda i, c: (c, 0)),
        ],
        out_specs=[
            pl.BlockSpec((rb, HC), lambda i, c: (c * (N // rb) + i, 0)),
            pl.BlockSpec((rb, 128), lambda i, c: (i, 0)),
        ],
        out_shape=[
            jax.ShapeDtypeStruct((NCH * N, HC), jnp.float32),
            jax.ShapeDtypeStruct((N, 128), jnp.float32),
        ],
    )(x, W1.reshape(DIN, NCH, HC).transpose(1, 0, 2),
      b1.reshape(NCH, 1, HC), aa)


EPW16 = EPW + 16 - EPW % 16 if EPW % 16 else EPW


@functools.partial(
    pl.kernel,
    out_type=(jax.ShapeDtypeStruct((E,), jnp.float32),
              jax.ShapeDtypeStruct((NW * N,), jnp.float32)),
    mesh=_mesh,
    compiler_params=pltpu.CompilerParams(needs_layout_passes=False),
    scratch_types=[
        pltpu.VMEM((N,), jnp.float32),
        pltpu.VMEM((N,), jnp.float32),
        pltpu.VMEM((N,), jnp.float32),
        pltpu.VMEM((EPW16,), jnp.int32),
        pltpu.VMEM((EPW16,), jnp.int32),
        pltpu.VMEM((EPW16,), jnp.float32),
    ],
)
def _edge_u(es_hbm, ed_hbm, src_hbm, dst_hbm, u_hbm, den_hbm,
            es_v, ed_v, den_v, src_v, dst_v, u_v):
    cid = lax.axis_index("c")
    sid = lax.axis_index("s")
    wid = sid * NC + cid
    base = pl.multiple_of(wid * EPW, 8)
    pltpu.sync_copy(es_hbm, es_v)
    pltpu.sync_copy(ed_hbm, ed_v)
    pltpu.sync_copy(src_hbm.at[pl.ds(base, EPW)], src_v.at[pl.ds(0, EPW)])
    pltpu.sync_copy(dst_hbm.at[pl.ds(base, EPW)], dst_v.at[pl.ds(0, EPW)])

    def zero_body(i, _):
        den_v[pl.ds(i * 16, 16)] = jnp.zeros((16,), jnp.float32)
        return 0

    lax.fori_loop(0, N // 16, zero_body, 0)

    lane = lax.iota(jnp.int32, 16)

    def edge_body(t, _):
        off = t * 16
        valid = (off + lane) < EPW
        s16 = jnp.where(valid, src_v[pl.ds(off, 16)], 0)
        d16 = jnp.where(valid, dst_v[pl.ds(off, 16)], 0)
        a = plsc.load_gather(es_v, [s16])
        b = plsc.load_gather(ed_v, [d16])
        e16 = a + b
        e16 = jnp.where(e16 >= 0, e16, 0.2 * e16)
        u16 = jnp.where(valid, jnp.exp(e16), 0.0)
        u_v[pl.ds(off, 16)] = u16
        plsc.addupdate_scatter(den_v, [d16], u16)
        return 0

    lax.fori_loop(0, EPW16 // 16, edge_body, 0)

    pltpu.sync_copy(u_v.at[pl.ds(0, EPW)], u_hbm.at[pl.ds(base, EPW)])
    dbase = pl.multiple_of(wid * N, 8)
    pltpu.sync_copy(den_v, den_hbm.at[pl.ds(dbase, N)])


@functools.partial(
    pl.kernel,
    out_type=jax.ShapeDtypeStruct((NCH * N, HC), jnp.float32),
    mesh=_mesh,
    compiler_params=pltpu.CompilerParams(needs_layout_passes=False),
    scratch_types=[
        pltpu.VMEM((EPT,), jnp.int32),
        pltpu.VMEM((NBUF, CB, HC), jnp.float32),
        pltpu.VMEM((NBUF, CB), jnp.int32),
        pltpu.VMEM((NBUF, CB), jnp.float32),
        pltpu.VMEM_SHARED((NSH, HC), jnp.float32),
    ] + [pltpu.SemaphoreType.DMA] * NBUF,
)
def _aggregate(h4_hbm, u_hbm, src_hbm, dst_hbm, agg_hbm,
               src_v, rows_v, didxb_v, ub_v, acc_sh, *sems):
    cid = lax.axis_index("c")
    sid = lax.axis_index("s")
    ebase = pl.multiple_of(sid * EPT, 8)
    pltpu.sync_copy(src_hbm.at[pl.ds(ebase, EPT)], src_v)

    def shift_body(i, s):
        sl = pl.ds(i * 16, 16)
        src_v[sl] = src_v[sl] + s
        return s

    def fire(g, b):
        off = (g * NBUF + b) * CB
        pltpu.async_copy(
            h4_hbm.at[src_v.at[pl.ds(off, CB)]], rows_v.at[b], sems[b])
        pltpu.async_copy(dst_hbm.at[pl.ds(ebase + off, CB)],
                         didxb_v.at[b], sems[b])
        pltpu.async_copy(u_hbm.at[pl.ds(ebase + off, CB)],
                         ub_v.at[b], sems[b])

    def zfill(r, _):
        for j in range(HC // 16):
            rows_v[0, r, pl.ds(j * 16, 16)] = jnp.zeros((16,), jnp.float32)
        return 0

    for hc in range(NCH // NC):
        chunk = cid * (NCH // NC) + hc
        lax.fori_loop(0, EPT // 16, shift_body,
                      N if hc else chunk * N, unroll=4)
        lax.fori_loop(0, CB, zfill, 0)
        for k in range(SRT // CB):
            pltpu.sync_copy(rows_v.at[0],
                            acc_sh.at[pl.ds(sid * SRT + k * CB, CB)])
        plsc.subcore_barrier()

        for b in range(NBUF):
            fire(0, b)

        def group_body(g, _):
            for b in range(NBUF):
                off = (g * NBUF + b) * CB
                pltpu.make_async_copy(
                    h4_hbm.at[src_v.at[pl.ds(off, CB)]],
                    rows_v.at[b], sems[b]).wait()
                pltpu.make_async_copy(dst_hbm.at[pl.ds(ebase + off, CB)],
                                      didxb_v.at[b], sems[b]).wait()
                pltpu.make_async_copy(u_hbm.at[pl.ds(ebase + off, CB)],
                                      ub_v.at[b], sems[b]).wait()
                rb = rows_v.at[b]
                ub = ub_v.at[b]

                def scale_body(i, _2):
                    us = plsc.load_gather(ub, [jnp.full((16,), i, jnp.int32)])
                    for j in range(HC // 16):
                        sl = pl.ds(j * 16, 16)
                        rb[i, sl] = rb[i, sl] * us
                    return 0

                lax.fori_loop(0, CB, scale_body, 0)
                pltpu.sync_copy(rb, acc_sh.at[didxb_v.at[b]], add=True)

                @pl.when(g < NG - 1)
                def _():
                    fire(g + 1, b)
            return 0

        lax.fori_loop(0, NG, group_body, 0)
        plsc.subcore_barrier()
        last_rows = N - (NS - 1) * SRT

        @pl.when(sid < NS - 1)
        def _():
            pltpu.sync_copy(
                acc_sh.at[pl.ds(sid * SRT, SRT)],
                agg_hbm.at[pl.ds(chunk * N + sid * SRT, SRT)])

        @pl.when(sid == NS - 1)
        def _():
            pltpu.sync_copy(
                acc_sh.at[pl.ds((NS - 1) * SRT, last_rows)],
                agg_hbm.at[pl.ds(chunk * N + (NS - 1) * SRT, last_rows)])
        plsc.subcore_barrier()


def _proj_body(ag_ref, denp_ref, w1_ref, w2_ref, bp_ref, p1_ref, p2_ref):
    c = pl.program_id(1)
    den = jnp.sum(denp_ref[...], axis=1)
    inv = 1.0 / (den + 1e-9)
    sc = ag_ref[...] * inv[:, None]
    q1 = jnp.dot(sc, w1_ref[...], preferred_element_type=jnp.float32)
    q2 = jnp.dot(sc, w2_ref[...], preferred_element_type=jnp.float32)

    @pl.when(c == 0)
    def _():
        p1_ref[...] = q1 + bp_ref[...]
        p2_ref[...] = q2

    @pl.when(c != 0)
    def _():
        p1_ref[...] += q1
        p2_ref[...] += q2


def _proj(aggu, denp, Wp, bp):
    rb = 1000
    return pl.pallas_call(
        _proj_body,
        grid=(N // rb, NCH),
        in_specs=[
            pl.BlockSpec((rb, HC), lambda i, c: (c * (N // rb) + i, 0)),
            pl.BlockSpec((rb, NW), lambda i, c: (i, 0)),
            pl.BlockSpec((HC, NRP), lambda i, c: (c, 0)),
            pl.BlockSpec((HC, NRP), lambda i, c: (NCH + c, 0)),
            pl.BlockSpec((NRP,), lambda i, c: (0,)),
        ],
        out_specs=[
            pl.BlockSpec((rb, NRP), lambda i, c: (i, 0)),
            pl.BlockSpec((rb, NRP), lambda i, c: (i, 0)),
        ],
        out_shape=[
            jax.ShapeDtypeStruct((N, NRP), jnp.float32),
            jax.ShapeDtypeStruct((N, NRP), jnp.float32),
        ],
    )(aggu, denp, Wp, Wp, bp)


@functools.partial(
    pl.kernel,
    out_type=jax.ShapeDtypeStruct((E, NRP), jnp.float32),
    mesh=_mesh,
    compiler_params=pltpu.CompilerParams(needs_layout_passes=False),
    scratch_types=[
        pltpu.VMEM((EB,), jnp.int32),
        pltpu.VMEM((EB,), jnp.int32),
        pltpu.VMEM((EB, NRP), jnp.float32),
        pltpu.VMEM((EB, NRP), jnp.float32),
        pltpu.SemaphoreType.DMA,
    ],
)
def _edge_out(p1_hbm, p2_hbm, src_hbm, dst_hbm, lp_hbm,
              i1_v, i2_v, a_v, b_v, sem):
    cid = lax.axis_index("c")
    sid = lax.axis_index("s")
    wid = sid * NC + cid
    base = wid * EPW

    def batch_body(t, _):
        eb = pl.multiple_of(base + t * EB, 8)
        pltpu.sync_copy(src_hbm.at[pl.ds(eb, EB)], i1_v)
        pltpu.sync_copy(dst_hbm.at[pl.ds(eb, EB)], i2_v)
        pltpu.async_copy(p1_hbm.at[i1_v], a_v, sem).wait()
        pltpu.async_copy(p2_hbm.at[i2_v], b_v, sem).wait()

        def add_body(i, _2):
            for j in range(NRP // 16):
                sl = pl.ds(j * 16, 16)
                a_v[i, sl] = a_v[i, sl] + b_v[i, sl]
            return 0

        lax.fori_loop(0, EB, add_body, 0)
        pltpu.sync_copy(a_v, lp_hbm.at[pl.ds(eb, EB)])
        return 0

    lax.fori_loop(0, EPW // EB, batch_body, 0)


def kernel(x, W1, b1, a_src, a_dst, W_edge, b_edge, edge_index):
    src = edge_index[0]
    dst = edge_index[1]
    aa = jnp.concatenate(
        [a_src[:, None], a_dst[:, None], jnp.zeros((H, 126), jnp.float32)],
        axis=1)
    h4, esd = _stage_a(x, W1, b1, aa)
    es = esd[:, 0]
    ed = esd[:, 1]
    u, denp = _edge_u(es, ed, src, dst)
    pad = EP - E
    srcp = jnp.concatenate([src, jnp.zeros((pad,), jnp.int32)])
    dstp = jnp.concatenate([dst, jnp.zeros((pad,), jnp.int32)])
    up = jnp.concatenate([u, jnp.zeros((pad,), jnp.float32)])
    aggu = _aggregate(h4, up, srcp, dstp)
    Wp = jnp.pad(W_edge, ((0, 0), (0, NRP - NR)))
    bp = jnp.pad(b_edge, (0, NRP - NR))
    P1, P2 = _proj(aggu, denp.reshape(NW, N).T, Wp, bp)
    logits_p = _edge_out(P1, P2, src, dst)
    return logits_p[:, :NR]


# speedup vs baseline: 4.8375x; 1.0251x over previous
import functools

import jax
import jax.numpy as jnp
from jax import lax
from jax.experimental import pallas as pl
from jax.experimental.pallas import tpu as pltpu
from jax.experimental.pallas import tpu_sc as plsc

N = 10000
E = 160000
DIN = 128
H = 768
NR = 47
NRP = 128

NC, NS = 2, 16
NW = NC * NS
HC = 128
NCH = H // HC
EPW = E // NW
EP = 163840
EPT = EP // NS
CB = 128
NBUF = 2
NB = EPT // CB
NG = NB // NBUF
NSH = 10240
SRT = NSH // NS
EPP = EP // NW
EB = 128
NEB = EPP // EB
NEG = NEB // NBUF

_mesh = plsc.VectorSubcoreMesh(core_axis_name="c", subcore_axis_name="s")


def _stage_a_body(x_ref, w1_ref, b1_ref, aa_ref, h4_ref, esd_ref):
    c = pl.program_id(1)
    hb = jax.nn.gelu(
        jnp.dot(x_ref[...], w1_ref[0], preferred_element_type=jnp.float32)
        + b1_ref[0])
    h4_ref[...] = hb
    pe = jnp.dot(hb, aa_ref[...], preferred_element_type=jnp.float32)

    @pl.when(c == 0)
    def _():
        esd_ref[...] = pe

    @pl.when(c != 0)
    def _():
        esd_ref[...] += pe


def _stage_a(x, W1, b1, aa):
    rb = 1000
    return pl.pallas_call(
        _stage_a_body,
        grid=(N // rb, NCH),
        in_specs=[
            pl.BlockSpec((rb, DIN), lambda i, c: (i, 0)),
            pl.BlockSpec((1, DIN, HC), lambda i, c: (c, 0, 0)),
            pl.BlockSpec((1, 1, HC), lambda i, c: (c, 0, 0)),
            pl.BlockSpec((HC, 128), lambda i, c: (c, 0)),
        ],
        out_specs=[
            pl.BlockSpec((rb, HC), lambda i, c: (c * (N // rb) + i, 0)),
            pl.BlockSpec((rb, 128), lambda i, c: (i, 0)),
        ],
        out_shape=[
            jax.ShapeDtypeStruct((NCH * N, HC), jnp.float32),
            jax.ShapeDtypeStruct((N, 128), jnp.float32),
        ],
    )(x, W1.reshape(DIN, NCH, HC).transpose(1, 0, 2),
      b1.reshape(NCH, 1, HC), aa)


EPW16 = EPW + 16 - EPW % 16 if EPW % 16 else EPW


@functools.partial(
    pl.kernel,
    out_type=(jax.ShapeDtypeStruct((E,), jnp.float32),
              jax.ShapeDtypeStruct((NW * N,), jnp.float32)),
    mesh=_mesh,
    compiler_params=pltpu.CompilerParams(needs_layout_passes=False),
    scratch_types=[
        pltpu.VMEM((N,), jnp.float32),
        pltpu.VMEM((N,), jnp.float32),
        pltpu.VMEM((N,), jnp.float32),
        pltpu.VMEM((EPW16,), jnp.int32),
        pltpu.VMEM((EPW16,), jnp.int32),
        pltpu.VMEM((EPW16,), jnp.float32),
    ],
)
def _edge_u(es_hbm, ed_hbm, src_hbm, dst_hbm, u_hbm, den_hbm,
            es_v, ed_v, den_v, src_v, dst_v, u_v):
    cid = lax.axis_index("c")
    sid = lax.axis_index("s")
    wid = sid * NC + cid
    base = pl.multiple_of(wid * EPW, 8)
    pltpu.sync_copy(es_hbm, es_v)
    pltpu.sync_copy(ed_hbm, ed_v)
    pltpu.sync_copy(src_hbm.at[pl.ds(base, EPW)], src_v.at[pl.ds(0, EPW)])
    pltpu.sync_copy(dst_hbm.at[pl.ds(base, EPW)], dst_v.at[pl.ds(0, EPW)])

    def zero_body(i, _):
        den_v[pl.ds(i * 16, 16)] = jnp.zeros((16,), jnp.float32)
        return 0

    lax.fori_loop(0, N // 16, zero_body, 0)

    lane = lax.iota(jnp.int32, 16)

    def edge_body(t, _):
        off = t * 16
        valid = (off + lane) < EPW
        s16 = jnp.where(valid, src_v[pl.ds(off, 16)], 0)
        d16 = jnp.where(valid, dst_v[pl.ds(off, 16)], 0)
        a = plsc.load_gather(es_v, [s16])
        b = plsc.load_gather(ed_v, [d16])
        e16 = a + b
        e16 = jnp.where(e16 >= 0, e16, 0.2 * e16)
        u16 = jnp.where(valid, jnp.exp(e16), 0.0)
        u_v[pl.ds(off, 16)] = u16
        plsc.addupdate_scatter(den_v, [d16], u16)
        return 0

    lax.fori_loop(0, EPW16 // 16, edge_body, 0)

    pltpu.sync_copy(u_v.at[pl.ds(0, EPW)], u_hbm.at[pl.ds(base, EPW)])
    dbase = pl.multiple_of(wid * N, 8)
    pltpu.sync_copy(den_v, den_hbm.at[pl.ds(dbase, N)])


@functools.partial(
    pl.kernel,
    out_type=jax.ShapeDtypeStruct((NCH * N, HC), jnp.float32),
    mesh=_mesh,
    compiler_params=pltpu.CompilerParams(needs_layout_passes=False),
    scratch_types=[
        pltpu.VMEM((EPT,), jnp.int32),
        pltpu.VMEM((NBUF, CB, HC), jnp.float32),
        pltpu.VMEM((NBUF, CB), jnp.int32),
        pltpu.VMEM((NBUF, CB), jnp.float32),
        pltpu.VMEM_SHARED((NSH, HC), jnp.float32),
    ] + [pltpu.SemaphoreType.DMA] * NBUF,
)
def _aggregate(h4_hbm, u_hbm, src_hbm, dst_hbm, agg_hbm,
               src_v, rows_v, didxb_v, ub_v, acc_sh, *sems):
    cid = lax.axis_index("c")
    sid = lax.axis_index("s")
    ebase = pl.multiple_of(sid * EPT, 8)
    pltpu.sync_copy(src_hbm.at[pl.ds(ebase, EPT)], src_v)

    def shift_body(i, s):
        sl = pl.ds(i * 16, 16)
        src_v[sl] = src_v[sl] + s
        return s

    def fire(g, b):
        off = (g * NBUF + b) * CB
        pltpu.async_copy(
            h4_hbm.at[src_v.at[pl.ds(off, CB)]], rows_v.at[b], sems[b])
        pltpu.async_copy(dst_hbm.at[pl.ds(ebase + off, CB)],
                         didxb_v.at[b], sems[b])
        pltpu.async_copy(u_hbm.at[pl.ds(ebase + off, CB)],
                         ub_v.at[b], sems[b])

    def zfill(r, _):
        for j in range(HC // 16):
            rows_v[0, r, pl.ds(j * 16, 16)] = jnp.zeros((16,), jnp.float32)
        return 0

    for hc in range(NCH // NC):
        chunk = cid * (NCH // NC) + hc
        lax.fori_loop(0, EPT // 16, shift_body,
                      N if hc else chunk * N, unroll=4)
        lax.fori_loop(0, CB, zfill, 0)
        for k in range(SRT // CB):
            pltpu.sync_copy(rows_v.at[0],
                            acc_sh.at[pl.ds(sid * SRT + k * CB, CB)])
        plsc.subcore_barrier()

        for b in range(NBUF):
            fire(0, b)

        def group_body(g, _):
            for b in range(NBUF):
                off = (g * NBUF + b) * CB
                pltpu.make_async_copy(
                    h4_hbm.at[src_v.at[pl.ds(off, CB)]],
                    rows_v.at[b], sems[b]).wait()
                pltpu.make_async_copy(dst_hbm.at[pl.ds(ebase + off, CB)],
                                      didxb_v.at[b], sems[b]).wait()
                pltpu.make_async_copy(u_hbm.at[pl.ds(ebase + off, CB)],
                                      ub_v.at[b], sems[b]).wait()
                rb = rows_v.at[b]
                ub = ub_v.at[b]

                def scale_body(i, _2):
                    us = plsc.load_gather(ub, [jnp.full((16,), i, jnp.int32)])
                    for j in range(HC // 16):
                        sl = pl.ds(j * 16, 16)
                        rb[i, sl] = rb[i, sl] * us
                    return 0

                lax.fori_loop(0, CB, scale_body, 0, unroll=4)
                pltpu.sync_copy(rb, acc_sh.at[didxb_v.at[b]], add=True)

                @pl.when(g < NG - 1)
                def _():
                    fire(g + 1, b)
            return 0

        lax.fori_loop(0, NG, group_body, 0)
        plsc.subcore_barrier()
        last_rows = N - (NS - 1) * SRT

        @pl.when(sid < NS - 1)
        def _():
            pltpu.sync_copy(
                acc_sh.at[pl.ds(sid * SRT, SRT)],
                agg_hbm.at[pl.ds(chunk * N + sid * SRT, SRT)])

        @pl.when(sid == NS - 1)
        def _():
            pltpu.sync_copy(
                acc_sh.at[pl.ds((NS - 1) * SRT, last_rows)],
                agg_hbm.at[pl.ds(chunk * N + (NS - 1) * SRT, last_rows)])
        plsc.subcore_barrier()


def _proj_body(ag_ref, denp_ref, w1_ref, w2_ref, bp_ref, p1_ref, p2_ref):
    c = pl.program_id(1)
    den = jnp.sum(denp_ref[...], axis=1)
    inv = 1.0 / (den + 1e-9)
    sc = ag_ref[...] * inv[:, None]
    q1 = jnp.dot(sc, w1_ref[...], preferred_element_type=jnp.float32)
    q2 = jnp.dot(sc, w2_ref[...], preferred_element_type=jnp.float32)

    @pl.when(c == 0)
    def _():
        p1_ref[...] = q1 + bp_ref[...]
        p2_ref[...] = q2

    @pl.when(c != 0)
    def _():
        p1_ref[...] += q1
        p2_ref[...] += q2


def _proj(aggu, denp, Wp, bp):
    rb = 1000
    return pl.pallas_call(
        _proj_body,
        grid=(N // rb, NCH),
        in_specs=[
            pl.BlockSpec((rb, HC), lambda i, c: (c * (N // rb) + i, 0)),
            pl.BlockSpec((rb, NW), lambda i, c: (i, 0)),
            pl.BlockSpec((HC, NRP), lambda i, c: (c, 0)),
            pl.BlockSpec((HC, NRP), lambda i, c: (NCH + c, 0)),
            pl.BlockSpec((NRP,), lambda i, c: (0,)),
        ],
        out_specs=[
            pl.BlockSpec((rb, NRP), lambda i, c: (i, 0)),
            pl.BlockSpec((rb, NRP), lambda i, c: (i, 0)),
        ],
        out_shape=[
            jax.ShapeDtypeStruct((N, NRP), jnp.float32),
            jax.ShapeDtypeStruct((N, NRP), jnp.float32),
        ],
    )(aggu, denp, Wp, Wp, bp)


@functools.partial(
    pl.kernel,
    out_type=jax.ShapeDtypeStruct((EP, NRP), jnp.float32),
    mesh=_mesh,
    compiler_params=pltpu.CompilerParams(needs_layout_passes=False),
    scratch_types=[
        pltpu.VMEM((EPP,), jnp.int32),
        pltpu.VMEM((EPP,), jnp.int32),
        pltpu.VMEM((NBUF, EB, NRP), jnp.float32),
        pltpu.VMEM((NBUF, EB, NRP), jnp.float32),
        pltpu.VMEM((NBUF, EB, NRP), jnp.float32),
    ] + [pltpu.SemaphoreType.DMA] * (2 * NBUF),
)
def _edge_out(p1_hbm, p2_hbm, src_hbm, dst_hbm, lp_hbm,
              i1_v, i2_v, a_v, b_v, o_v, *sems):
    gsems = sems[:NBUF]
    wsems = sems[NBUF:]
    cid = lax.axis_index("c")
    sid = lax.axis_index("s")
    wid = sid * NC + cid
    base = pl.multiple_of(wid * EPP, 8)
    pltpu.sync_copy(src_hbm.at[pl.ds(base, EPP)], i1_v)
    pltpu.sync_copy(dst_hbm.at[pl.ds(base, EPP)], i2_v)

    def fire(g, b):
        off = (g * NBUF + b) * EB
        pltpu.async_copy(p1_hbm.at[i1_v.at[pl.ds(off, EB)]],
                         a_v.at[b], gsems[b])
        pltpu.async_copy(p2_hbm.at[i2_v.at[pl.ds(off, EB)]],
                         b_v.at[b], gsems[b])

    for b in range(NBUF):
        fire(0, b)

    def group_body(g, _):
        for b in range(NBUF):
            off = (g * NBUF + b) * EB
            pltpu.make_async_copy(p1_hbm.at[i1_v.at[pl.ds(off, EB)]],
                                  a_v.at[b], gsems[b]).wait()
            pltpu.make_async_copy(p2_hbm.at[i2_v.at[pl.ds(off, EB)]],
                                  b_v.at[b], gsems[b]).wait()
            ab = a_v.at[b]
            bb = b_v.at[b]
            ob = o_v.at[b]

            @pl.when(g > 0)
            def _():
                pltpu.make_async_copy(
                    ob, lp_hbm.at[pl.ds(base, EB)], wsems[b]).wait()

            def add_body(i, _2):
                for j in range(NRP // 16):
                    sl = pl.ds(j * 16, 16)
                    ob[i, sl] = ab[i, sl] + bb[i, sl]
                return 0

            lax.fori_loop(0, EB, add_body, 0, unroll=4)
            pltpu.async_copy(ob, lp_hbm.at[pl.ds(base + off, EB)], wsems[b])

            @pl.when(g < NEG - 1)
            def _():
                fire(g + 1, b)
        return 0

    lax.fori_loop(0, NEG, group_body, 0)
    for b in range(NBUF):
        pltpu.make_async_copy(
            o_v.at[b], lp_hbm.at[pl.ds(base, EB)], wsems[b]).wait()


def kernel(x, W1, b1, a_src, a_dst, W_edge, b_edge, edge_index):
    src = edge_index[0]
    dst = edge_index[1]
    aa = jnp.concatenate(
        [a_src[:, None], a_dst[:, None], jnp.zeros((H, 126), jnp.float32)],
        axis=1)
    h4, esd = _stage_a(x, W1, b1, aa)
    es = esd[:, 0]
    ed = esd[:, 1]
    u, denp = _edge_u(es, ed, src, dst)
    pad = EP - E
    srcp = jnp.concatenate([src, jnp.zeros((pad,), jnp.int32)])
    dstp = jnp.concatenate([dst, jnp.zeros((pad,), jnp.int32)])
    up = jnp.concatenate([u, jnp.zeros((pad,), jnp.float32)])
    aggu = _aggregate(h4, up, srcp, dstp)
    Wp = jnp.pad(W_edge, ((0, 0), (0, NRP - NR)))
    bp = jnp.pad(b_edge, (0, NRP - NR))
    P1, P2 = _proj(aggu, denp.reshape(NW, N).T, Wp, bp)
    logits_p = _edge_out(P1, P2, srcp, dstp)
    return logits_p[:E, :NR]


# speedup vs baseline: 4.9114x; 1.0153x over previous
import functools

import jax
import jax.numpy as jnp
from jax import lax
from jax.experimental import pallas as pl
from jax.experimental.pallas import tpu as pltpu
from jax.experimental.pallas import tpu_sc as plsc

N = 10000
E = 160000
DIN = 128
H = 768
NR = 47
NRP = 128

NC, NS = 2, 16
NW = NC * NS
HC = 128
NCH = H // HC
EPW = E // NW
EP = 163840
EPT = EP // NS
CB = 64
CBUF = 4
NB = EPT // CB
NG = NB // CBUF
NBUF = 2
NSH = 10240
SRT = NSH // NS
EPP = EP // NW
EB = 128
NEB = EPP // EB
NEG = NEB // NBUF

_mesh = plsc.VectorSubcoreMesh(core_axis_name="c", subcore_axis_name="s")


def _stage_a_body(x_ref, w1_ref, b1_ref, aa_ref, h4_ref, esd_ref):
    c = pl.program_id(1)
    hb = jax.nn.gelu(
        jnp.dot(x_ref[...], w1_ref[0], preferred_element_type=jnp.float32)
        + b1_ref[0])
    h4_ref[...] = hb
    pe = jnp.dot(hb, aa_ref[...], preferred_element_type=jnp.float32)

    @pl.when(c == 0)
    def _():
        esd_ref[...] = pe

    @pl.when(c != 0)
    def _():
        esd_ref[...] += pe


def _stage_a(x, W1, b1, aa):
    rb = 1000
    return pl.pallas_call(
        _stage_a_body,
        grid=(N // rb, NCH),
        in_specs=[
            pl.BlockSpec((rb, DIN), lambda i, c: (i, 0)),
            pl.BlockSpec((1, DIN, HC), lambda i, c: (c, 0, 0)),
            pl.BlockSpec((1, 1, HC), lambda i, c: (c, 0, 0)),
            pl.BlockSpec((HC, 128), lambda i, c: (c, 0)),
        ],
        out_specs=[
            pl.BlockSpec((rb, HC), lambda i, c: (c * (N // rb) + i, 0)),
            pl.BlockSpec((rb, 128), lambda i, c: (i, 0)),
        ],
        out_shape=[
            jax.ShapeDtypeStruct((NCH * N, HC), jnp.float32),
            jax.ShapeDtypeStruct((N, 128), jnp.float32),
        ],
    )(x, W1.reshape(DIN, NCH, HC).transpose(1, 0, 2),
      b1.reshape(NCH, 1, HC), aa)


EPW16 = EPW + 16 - EPW % 16 if EPW % 16 else EPW


@functools.partial(
    pl.kernel,
    out_type=(jax.ShapeDtypeStruct((E,), jnp.float32),
              jax.ShapeDtypeStruct((NW * N,), jnp.float32)),
    mesh=_mesh,
    compiler_params=pltpu.CompilerParams(needs_layout_passes=False),
    scratch_types=[
        pltpu.VMEM((N,), jnp.float32),
        pltpu.VMEM((N,), jnp.float32),
        pltpu.VMEM((N,), jnp.float32),
        pltpu.VMEM((EPW16,), jnp.int32),
        pltpu.VMEM((EPW16,), jnp.int32),
        pltpu.VMEM((EPW16,), jnp.float32),
    ],
)
def _edge_u(es_hbm, ed_hbm, src_hbm, dst_hbm, u_hbm, den_hbm,
            es_v, ed_v, den_v, src_v, dst_v, u_v):
    cid = lax.axis_index("c")
    sid = lax.axis_index("s")
    wid = sid * NC + cid
    base = pl.multiple_of(wid * EPW, 8)
    pltpu.sync_copy(es_hbm, es_v)
    pltpu.sync_copy(ed_hbm, ed_v)
    pltpu.sync_copy(src_hbm.at[pl.ds(base, EPW)], src_v.at[pl.ds(0, EPW)])
    pltpu.sync_copy(dst_hbm.at[pl.ds(base, EPW)], dst_v.at[pl.ds(0, EPW)])

    def zero_body(i, _):
        den_v[pl.ds(i * 16, 16)] = jnp.zeros((16,), jnp.float32)
        return 0

    lax.fori_loop(0, N // 16, zero_body, 0)

    lane = lax.iota(jnp.int32, 16)

    def edge_body(t, _):
        off = t * 16
        valid = (off + lane) < EPW
        s16 = jnp.where(valid, src_v[pl.ds(off, 16)], 0)
        d16 = jnp.where(valid, dst_v[pl.ds(off, 16)], 0)
        a = plsc.load_gather(es_v, [s16])
        b = plsc.load_gather(ed_v, [d16])
        e16 = a + b
        e16 = jnp.where(e16 >= 0, e16, 0.2 * e16)
        u16 = jnp.where(valid, jnp.exp(e16), 0.0)
        u_v[pl.ds(off, 16)] = u16
        plsc.addupdate_scatter(den_v, [d16], u16)
        return 0

    lax.fori_loop(0, EPW16 // 16, edge_body, 0)

    pltpu.sync_copy(u_v.at[pl.ds(0, EPW)], u_hbm.at[pl.ds(base, EPW)])
    dbase = pl.multiple_of(wid * N, 8)
    pltpu.sync_copy(den_v, den_hbm.at[pl.ds(dbase, N)])


@functools.partial(
    pl.kernel,
    out_type=jax.ShapeDtypeStruct((NCH * N, HC), jnp.float32),
    mesh=_mesh,
    compiler_params=pltpu.CompilerParams(needs_layout_passes=False),
    scratch_types=[
        pltpu.VMEM((EPT,), jnp.int32),
        pltpu.VMEM((CBUF, CB, HC), jnp.float32),
        pltpu.VMEM((CBUF, CB), jnp.int32),
        pltpu.VMEM((CBUF, CB), jnp.float32),
        pltpu.VMEM_SHARED((NSH, HC), jnp.float32),
    ] + [pltpu.SemaphoreType.DMA] * (2 * CBUF),
)
def _aggregate(h4_hbm, u_hbm, src_hbm, dst_hbm, agg_hbm,
               src_v, rows_v, didxb_v, ub_v, acc_sh, *sems):
    gsems = sems[:CBUF]
    ssems = sems[CBUF:]
    cid = lax.axis_index("c")
    sid = lax.axis_index("s")
    ebase = pl.multiple_of(sid * EPT, 8)
    pltpu.sync_copy(src_hbm.at[pl.ds(ebase, EPT)], src_v)

    def shift_body(i, s):
        sl = pl.ds(i * 16, 16)
        src_v[sl] = src_v[sl] + s
        return s

    def fire(t, b):
        off = t * CB
        pltpu.async_copy(
            h4_hbm.at[src_v.at[pl.ds(off, CB)]], rows_v.at[b], gsems[b])
        pltpu.async_copy(dst_hbm.at[pl.ds(ebase + off, CB)],
                         didxb_v.at[b], gsems[b])
        pltpu.async_copy(u_hbm.at[pl.ds(ebase + off, CB)],
                         ub_v.at[b], gsems[b])

    def drain_scatter(b):
        pltpu.make_async_copy(
            rows_v.at[b], acc_sh.at[didxb_v.at[b]], ssems[b]).wait()

    def zfill(r, _):
        for j in range(HC // 16):
            rows_v[0, r, pl.ds(j * 16, 16)] = jnp.zeros((16,), jnp.float32)
        return 0

    for hc in range(NCH // NC):
        chunk = cid * (NCH // NC) + hc
        lax.fori_loop(0, EPT // 16, shift_body,
                      N if hc else chunk * N, unroll=4)
        lax.fori_loop(0, CB, zfill, 0)
        for k in range(SRT // CB):
            pltpu.sync_copy(rows_v.at[0],
                            acc_sh.at[pl.ds(sid * SRT + k * CB, CB)])
        plsc.subcore_barrier()

        for b in range(2):
            fire(b, b)

        def group_body(g, _):
            for b in range(CBUF):
                t = g * CBUF + b
                pltpu.make_async_copy(
                    h4_hbm.at[src_v.at[pl.ds(t * CB, CB)]],
                    rows_v.at[b], gsems[b]).wait()
                pltpu.make_async_copy(dst_hbm.at[pl.ds(ebase + t * CB, CB)],
                                      didxb_v.at[b], gsems[b]).wait()
                pltpu.make_async_copy(u_hbm.at[pl.ds(ebase + t * CB, CB)],
                                      ub_v.at[b], gsems[b]).wait()
                rb = rows_v.at[b]
                ub = ub_v.at[b]

                def scale_body(i, _2):
                    us = plsc.load_gather(ub, [jnp.full((16,), i, jnp.int32)])
                    for j in range(HC // 16):
                        sl = pl.ds(j * 16, 16)
                        rb[i, sl] = rb[i, sl] * us
                    return 0

                lax.fori_loop(0, CB, scale_body, 0, unroll=4)
                b2 = (b + 2) % CBUF
                if b < 2:
                    @pl.when(g > 0)
                    def _():
                        drain_scatter(b2)
                    fire(t + 2, b2)
                else:
                    drain_scatter(b2)

                    @pl.when(g < NG - 1)
                    def _():
                        fire(t + 2, b2)
                pltpu.async_copy(rb, acc_sh.at[didxb_v.at[b]], ssems[b],
                                 add=True)
            return 0

        lax.fori_loop(0, NG, group_body, 0)
        for b in range(2, CBUF):
            drain_scatter(b)
        plsc.subcore_barrier()
        last_rows = N - (NS - 1) * SRT

        @pl.when(sid < NS - 1)
        def _():
            pltpu.sync_copy(
                acc_sh.at[pl.ds(sid * SRT, SRT)],
                agg_hbm.at[pl.ds(chunk * N + sid * SRT, SRT)])

        @pl.when(sid == NS - 1)
        def _():
            pltpu.sync_copy(
                acc_sh.at[pl.ds((NS - 1) * SRT, last_rows)],
                agg_hbm.at[pl.ds(chunk * N + (NS - 1) * SRT, last_rows)])
        plsc.subcore_barrier()


def _proj_body(ag_ref, denp_ref, w1_ref, w2_ref, bp_ref, p1_ref, p2_ref):
    c = pl.program_id(1)
    den = jnp.sum(denp_ref[...], axis=1)
    inv = 1.0 / (den + 1e-9)
    sc = ag_ref[...] * inv[:, None]
    q1 = jnp.dot(sc, w1_ref[...], preferred_element_type=jnp.float32)
    q2 = jnp.dot(sc, w2_ref[...], preferred_element_type=jnp.float32)

    @pl.when(c == 0)
    def _():
        p1_ref[...] = q1 + bp_ref[...]
        p2_ref[...] = q2

    @pl.when(c != 0)
    def _():
        p1_ref[...] += q1
        p2_ref[...] += q2


def _proj(aggu, denp, Wp, bp):
    rb = 1000
    return pl.pallas_call(
        _proj_body,
        grid=(N // rb, NCH),
        in_specs=[
            pl.BlockSpec((rb, HC), lambda i, c: (c * (N // rb) + i, 0)),
            pl.BlockSpec((rb, NW), lambda i, c: (i, 0)),
            pl.BlockSpec((HC, NRP), lambda i, c: (c, 0)),
            pl.BlockSpec((HC, NRP), lambda i, c: (NCH + c, 0)),
            pl.BlockSpec((NRP,), lambda i, c: (0,)),
        ],
        out_specs=[
            pl.BlockSpec((rb, NRP), lambda i, c: (i, 0)),
            pl.BlockSpec((rb, NRP), lambda i, c: (i, 0)),
        ],
        out_shape=[
            jax.ShapeDtypeStruct((N, NRP), jnp.float32),
            jax.ShapeDtypeStruct((N, NRP), jnp.float32),
        ],
    )(aggu, denp, Wp, Wp, bp)


@functools.partial(
    pl.kernel,
    out_type=jax.ShapeDtypeStruct((EP, NRP), jnp.float32),
    mesh=_mesh,
    compiler_params=pltpu.CompilerParams(needs_layout_passes=False),
    scratch_types=[
        pltpu.VMEM((EPP,), jnp.int32),
        pltpu.VMEM((EPP,), jnp.int32),
        pltpu.VMEM((NBUF, EB, NRP), jnp.float32),
        pltpu.VMEM((NBUF, EB, NRP), jnp.float32),
        pltpu.VMEM((NBUF, EB, NRP), jnp.float32),
    ] + [pltpu.SemaphoreType.DMA] * (2 * NBUF),
)
def _edge_out(p1_hbm, p2_hbm, src_hbm, dst_hbm, lp_hbm,
              i1_v, i2_v, a_v, b_v, o_v, *sems):
    gsems = sems[:NBUF]
    wsems = sems[NBUF:]
    cid = lax.axis_index("c")
    sid = lax.axis_index("s")
    wid = sid * NC + cid
    base = pl.multiple_of(wid * EPP, 8)
    pltpu.sync_copy(src_hbm.at[pl.ds(base, EPP)], i1_v)
    pltpu.sync_copy(dst_hbm.at[pl.ds(base, EPP)], i2_v)

    def fire(g, b):
        off = (g * NBUF + b) * EB
        pltpu.async_copy(p1_hbm.at[i1_v.at[pl.ds(off, EB)]],
                         a_v.at[b], gsems[b])
        pltpu.async_copy(p2_hbm.at[i2_v.at[pl.ds(off, EB)]],
                         b_v.at[b], gsems[b])

    for b in range(NBUF):
        fire(0, b)

    def group_body(g, _):
        for b in range(NBUF):
            off = (g * NBUF + b) * EB
            pltpu.make_async_copy(p1_hbm.at[i1_v.at[pl.ds(off, EB)]],
                                  a_v.at[b], gsems[b]).wait()
            pltpu.make_async_copy(p2_hbm.at[i2_v.at[pl.ds(off, EB)]],
                                  b_v.at[b], gsems[b]).wait()
            ab = a_v.at[b]
            bb = b_v.at[b]
            ob = o_v.at[b]

            @pl.when(g > 0)
            def _():
                pltpu.make_async_copy(
                    ob, lp_hbm.at[pl.ds(base, EB)], wsems[b]).wait()

            def add_body(i, _2):
                for j in range(NRP // 16):
                    sl = pl.ds(j * 16, 16)
                    ob[i, sl] = ab[i, sl] + bb[i, sl]
                return 0

            lax.fori_loop(0, EB, add_body, 0, unroll=4)
            pltpu.async_copy(ob, lp_hbm.at[pl.ds(base + off, EB)], wsems[b])

            @pl.when(g < NEG - 1)
            def _():
                fire(g + 1, b)
        return 0

    lax.fori_loop(0, NEG, group_body, 0)
    for b in range(NBUF):
        pltpu.make_async_copy(
            o_v.at[b], lp_hbm.at[pl.ds(base, EB)], wsems[b]).wait()


def kernel(x, W1, b1, a_src, a_dst, W_edge, b_edge, edge_index):
    src = edge_index[0]
    dst = edge_index[1]
    aa = jnp.concatenate(
        [a_src[:, None], a_dst[:, None], jnp.zeros((H, 126), jnp.float32)],
        axis=1)
    h4, esd = _stage_a(x, W1, b1, aa)
    es = esd[:, 0]
    ed = esd[:, 1]
    u, denp = _edge_u(es, ed, src, dst)
    pad = EP - E
    srcp = jnp.concatenate([src, jnp.zeros((pad,), jnp.int32)])
    dstp = jnp.concatenate([dst, jnp.zeros((pad,), jnp.int32)])
    up = jnp.concatenate([u, jnp.zeros((pad,), jnp.float32)])
    aggu = _aggregate(h4, up, srcp, dstp)
    Wp = jnp.pad(W_edge, ((0, 0), (0, NRP - NR)))
    bp = jnp.pad(b_edge, (0, NRP - NR))
    P1, P2 = _proj(aggu, denp.reshape(NW, N).T, Wp, bp)
    logits_p = _edge_out(P1, P2, srcp, dstp)
    return logits_p[:E, :NR]
